# baseline (speedup 1.0000x reference)
import functools

import jax
import jax.numpy as jnp
from jax import lax
from jax.experimental import pallas as pl
from jax.experimental.pallas import tpu as pltpu
from jax.experimental.pallas import tpu_sc as plsc

_H = 384
_W = 384
_B = 2
_C = 192
_NW = 32
_RPW = (_B * _H) // _NW
_CHUNK = 48
_GRP = _CHUNK // 16


def _floor_f32(x):
    t = x.astype(jnp.int32).astype(jnp.float32)
    return jnp.where(t > x, t - 1.0, t)


def _splat(ref, i):
    return plsc.load_gather(ref, [jnp.full((16,), i, jnp.int32)])


def _warp_body(src_t, dispx, dispy, xs, ys, out_t, flow_out,
               xs_v, ys_v, dx_v, dy_v,
               idx00, idx01, idx10, idx11,
               w00, w01, w10, w11,
               r00, r01, r10, r11,
               acc, flowbuf, sem):
    cid = lax.axis_index("c")
    sid = lax.axis_index("s")
    wid = sid * 2 + cid
    b = wid // 16
    row0 = wid * _RPW
    y0 = row0 - b * _H
    tbase = b * (_H * _W)

    pltpu.sync_copy(xs.at[:], xs_v)
    pltpu.sync_copy(ys.at[pl.ds(y0, _RPW)], ys_v)
    pltpu.sync_copy(dispx.at[pl.ds(row0, _RPW)], dx_v)
    pltpu.sync_copy(dispy.at[pl.ds(row0, _RPW)], dy_v)

    lanes = lax.iota(jnp.int32, 16)

    def chunk_body(q, _):
        j = q // 8
        c8 = q - j * 8
        yv = _splat(ys_v, j)
        for g in range(_GRP):
            x0 = c8 * _CHUNK + g * 16
            fx = xs_v[pl.ds(x0, 16)] + dx_v[j, pl.ds(x0, 16)]
            fy = yv + dy_v[j, pl.ds(x0, 16)]
            ix = (fx + 1.0) * 0.5 * float(_W - 1)
            iy = (fy + 1.0) * 0.5 * float(_H - 1)
            ix0 = _floor_f32(ix)
            iy0 = _floor_f32(iy)
            ix1 = ix0 + 1.0
            iy1 = iy0 + 1.0
            wx1 = ix - ix0
            wx0 = 1.0 - wx1
            wy1 = iy - iy0
            wy0 = 1.0 - wy1
            inx0 = (ix0 >= 0.0) & (ix0 <= float(_W - 1))
            inx1 = (ix1 >= 0.0) & (ix1 <= float(_W - 1))
            iny0 = (iy0 >= 0.0) & (iy0 <= float(_H - 1))
            iny1 = (iy1 >= 0.0) & (iy1 <= float(_H - 1))
            cx0 = jnp.clip(ix0, 0.0, float(_W - 1)).astype(jnp.int32)
            cx1 = jnp.clip(ix1, 0.0, float(_W - 1)).astype(jnp.int32)
            cy0 = jnp.clip(iy0, 0.0, float(_H - 1)).astype(jnp.int32) * _W + tbase
            cy1 = jnp.clip(iy1, 0.0, float(_H - 1)).astype(jnp.int32) * _W + tbase
            s = pl.ds(g * 16, 16)
            idx00[s] = cy0 + cx0
            idx01[s] = cy0 + cx1
            idx10[s] = cy1 + cx0
            idx11[s] = cy1 + cx1
            w00[s] = wy0 * wx0 * (iny0 & inx0).astype(jnp.float32)
            w01[s] = wy0 * wx1 * (iny0 & inx1).astype(jnp.float32)
            w10[s] = wy1 * wx0 * (iny1 & inx0).astype(jnp.float32)
            w11[s] = wy1 * wx1 * (iny1 & inx1).astype(jnp.float32)
            loc = (lanes + g * 16) * 2
            plsc.store_scatter(flowbuf, [loc], fx)
            plsc.store_scatter(flowbuf, [loc + 1], fy)

        c0 = pltpu.async_copy(src_t.at[idx00], r00, sem)
        c1 = pltpu.async_copy(src_t.at[idx01], r01, sem)
        c2 = pltpu.async_copy(src_t.at[idx10], r10, sem)
        c3 = pltpu.async_copy(src_t.at[idx11], r11, sem)
        c0.wait()
        c1.wait()
        c2.wait()
        c3.wait()

        def pix_body(i, _):
            b00 = _splat(w00, i)
            b01 = _splat(w01, i)
            b10 = _splat(w10, i)
            b11 = _splat(w11, i)
            for cc in range(_C // 16):
                cs = pl.ds(cc * 16, 16)
                acc[i, cs] = (r00[i, cs] * b00 + r01[i, cs] * b01
                              + r10[i, cs] * b10 + r11[i, cs] * b11)
            return _

        lax.fori_loop(0, _CHUNK, pix_body, None)

        outbase = (row0 + j) * _W + c8 * _CHUNK
        pltpu.sync_copy(acc, out_t.at[pl.ds(outbase, _CHUNK)])
        pltpu.sync_copy(flowbuf, flow_out.at[pl.ds(outbase * 2, _CHUNK * 2)])
        return _

    lax.fori_loop(0, _RPW * 8, chunk_body, None)


_warp = functools.partial(
    pl.kernel,
    out_type=(
        jax.ShapeDtypeStruct((_B * _H * _W, _C), jnp.float32),
        jax.ShapeDtypeStruct((_B * _H * _W * 2,), jnp.float32),
    ),
    mesh=plsc.VectorSubcoreMesh(core_axis_name="c", subcore_axis_name="s",
                                num_cores=2, num_subcores=16),
    compiler_params=pltpu.CompilerParams(needs_layout_passes=False,
                                         use_tc_tiling_on_sc=False),
    scratch_types=[
        pltpu.VMEM((_W,), jnp.float32),
        pltpu.VMEM((_RPW,), jnp.float32),
        pltpu.VMEM((_RPW, _W), jnp.float32),
        pltpu.VMEM((_RPW, _W), jnp.float32),
        pltpu.VMEM((_CHUNK,), jnp.int32),
        pltpu.VMEM((_CHUNK,), jnp.int32),
        pltpu.VMEM((_CHUNK,), jnp.int32),
        pltpu.VMEM((_CHUNK,), jnp.int32),
        pltpu.VMEM((_CHUNK,), jnp.float32),
        pltpu.VMEM((_CHUNK,), jnp.float32),
        pltpu.VMEM((_CHUNK,), jnp.float32),
        pltpu.VMEM((_CHUNK,), jnp.float32),
        pltpu.VMEM((_CHUNK, _C), jnp.float32),
        pltpu.VMEM((_CHUNK, _C), jnp.float32),
        pltpu.VMEM((_CHUNK, _C), jnp.float32),
        pltpu.VMEM((_CHUNK, _C), jnp.float32),
        pltpu.VMEM((_CHUNK, _C), jnp.float32),
        pltpu.VMEM((_CHUNK * 2,), jnp.float32),
        pltpu.SemaphoreType.DMA,
    ],
)(_warp_body)


def kernel(src, disp):
    src_t = src.transpose(0, 2, 3, 1).reshape(_B * _H * _W, _C)
    dispx = disp[:, 0].reshape(_B * _H, _W)
    dispy = disp[:, 1].reshape(_B * _H, _W)
    xs = jnp.linspace(-1.0, 1.0, _W, dtype=jnp.float32)
    ys = jnp.linspace(-1.0, 1.0, _H, dtype=jnp.float32)
    out_t, flow_flat = _warp(src_t, dispx, dispy, xs, ys)
    warped = out_t.reshape(_B, _H, _W, _C).transpose(0, 3, 1, 2)
    flow = flow_flat.reshape(_B, _H, _W, 2)
    return warped, flow


# speedup vs baseline: 1.1891x; 1.1891x over previous
import functools

import jax
import jax.numpy as jnp
from jax import lax
from jax.experimental import pallas as pl
from jax.experimental.pallas import tpu as pltpu
from jax.experimental.pallas import tpu_sc as plsc

_H = 384
_W = 384
_B = 2
_C = 192
_NW = 32
_RPW = (_B * _H) // _NW
_CHUNK = 48
_GRP = _CHUNK // 16
_NCHUNK = _RPW * 8


def _floor_f32(x):
    t = x.astype(jnp.int32).astype(jnp.float32)
    return jnp.where(t > x, t - 1.0, t)


def _splat(ref, i):
    return plsc.load_gather(ref, [jnp.full((16,), i, jnp.int32)])


def _warp_body(src_t, dispx, dispy, xs, ys, out_t, flow_out, *scr):
    xs_v, ys_v, dx_v, dy_v = scr[:4]
    sets = []
    for par in range(2):
        a = 4 + par * 12
        sets.append(dict(
            idx=scr[a:a + 4], w=scr[a + 4:a + 8], r=scr[a + 8:a + 12],
            acc=scr[28 + par], flow=scr[30 + par],
            gsem=scr[32 + par], osem=scr[34 + par],
        ))

    cid = lax.axis_index("c")
    sid = lax.axis_index("s")
    wid = sid * 2 + cid
    b = wid // 16
    row0 = wid * _RPW
    y0 = row0 - b * _H
    tbase = b * (_H * _W)

    pltpu.sync_copy(xs.at[:], xs_v)
    pltpu.sync_copy(ys.at[pl.ds(y0, _RPW)], ys_v)
    pltpu.sync_copy(dispx.at[pl.ds(row0, _RPW)], dx_v)
    pltpu.sync_copy(dispy.at[pl.ds(row0, _RPW)], dy_v)

    lanes = lax.iota(jnp.int32, 16)

    def fire(q, S):
        j = q // 8
        c8 = q - j * 8
        yv = _splat(ys_v, j)
        for g in range(_GRP):
            x0 = c8 * _CHUNK + g * 16
            fx = xs_v[pl.ds(x0, 16)] + dx_v[j, pl.ds(x0, 16)]
            fy = yv + dy_v[j, pl.ds(x0, 16)]
            ix = (fx + 1.0) * 0.5 * float(_W - 1)
            iy = (fy + 1.0) * 0.5 * float(_H - 1)
            ix0 = _floor_f32(ix)
            iy0 = _floor_f32(iy)
            ix1 = ix0 + 1.0
            iy1 = iy0 + 1.0
            wx1 = ix - ix0
            wx0 = 1.0 - wx1
            wy1 = iy - iy0
            wy0 = 1.0 - wy1
            inx0 = (ix0 >= 0.0) & (ix0 <= float(_W - 1))
            inx1 = (ix1 >= 0.0) & (ix1 <= float(_W - 1))
            iny0 = (iy0 >= 0.0) & (iy0 <= float(_H - 1))
            iny1 = (iy1 >= 0.0) & (iy1 <= float(_H - 1))
            cx0 = jnp.clip(ix0, 0.0, float(_W - 1)).astype(jnp.int32)
            cx1 = jnp.clip(ix1, 0.0, float(_W - 1)).astype(jnp.int32)
            cy0 = jnp.clip(iy0, 0.0, float(_H - 1)).astype(jnp.int32) * _W + tbase
            cy1 = jnp.clip(iy1, 0.0, float(_H - 1)).astype(jnp.int32) * _W + tbase
            s = pl.ds(g * 16, 16)
            S["idx"][0][s] = cy0 + cx0
            S["idx"][1][s] = cy0 + cx1
            S["idx"][2][s] = cy1 + cx0
            S["idx"][3][s] = cy1 + cx1
            S["w"][0][s] = wy0 * wx0 * (iny0 & inx0).astype(jnp.float32)
            S["w"][1][s] = wy0 * wx1 * (iny0 & inx1).astype(jnp.float32)
            S["w"][2][s] = wy1 * wx0 * (iny1 & inx0).astype(jnp.float32)
            S["w"][3][s] = wy1 * wx1 * (iny1 & inx1).astype(jnp.float32)
            loc = (lanes + g * 16) * 2
            plsc.store_scatter(S["flow"], [loc], fx)
            plsc.store_scatter(S["flow"], [loc + 1], fy)
        for t in range(4):
            pltpu.async_copy(src_t.at[S["idx"][t]], S["r"][t], S["gsem"])

    def drain_gathers(S):
        for t in range(4):
            pltpu.make_async_copy(src_t.at[S["idx"][t]], S["r"][t],
                                  S["gsem"]).wait()

    def blend(S):
        def pix_body(i, carry):
            b00 = _splat(S["w"][0], i)
            b01 = _splat(S["w"][1], i)
            b10 = _splat(S["w"][2], i)
            b11 = _splat(S["w"][3], i)
            for cc in range(_C // 16):
                cs = pl.ds(cc * 16, 16)
                S["acc"][i, cs] = (
                    S["r"][0][i, cs] * b00 + S["r"][1][i, cs] * b01
                    + S["r"][2][i, cs] * b10 + S["r"][3][i, cs] * b11)
            return carry

        lax.fori_loop(0, _CHUNK, pix_body, None)

    def write_out(q, S):
        base = (row0 + q // 8) * _W + (q - (q // 8) * 8) * _CHUNK
        pltpu.sync_copy(S["acc"], out_t.at[pl.ds(base, _CHUNK)])
        pltpu.sync_copy(S["flow"], flow_out.at[pl.ds(base * 2, _CHUNK * 2)])

    def loop_i(i, carry):
        fire(2 * i, sets[0])

        @pl.when(i > 0)
        def _():
            drain_gathers(sets[1])
            blend(sets[1])
            write_out(2 * i - 1, sets[1])

        fire(2 * i + 1, sets[1])
        drain_gathers(sets[0])
        blend(sets[0])
        write_out(2 * i, sets[0])
        return carry

    lax.fori_loop(0, _NCHUNK // 2, loop_i, None)
    drain_gathers(sets[1])
    blend(sets[1])
    write_out(_NCHUNK - 1, sets[1])


_scratch = [
    pltpu.VMEM((_W,), jnp.float32),
    pltpu.VMEM((_RPW,), jnp.float32),
    pltpu.VMEM((_RPW, _W), jnp.float32),
    pltpu.VMEM((_RPW, _W), jnp.float32),
]
for _par in range(2):
    _scratch += [pltpu.VMEM((_CHUNK,), jnp.int32) for _ in range(4)]
    _scratch += [pltpu.VMEM((_CHUNK,), jnp.float32) for _ in range(4)]
    _scratch += [pltpu.VMEM((_CHUNK, _C), jnp.float32) for _ in range(4)]
_scratch += [pltpu.VMEM((_CHUNK, _C), jnp.float32) for _ in range(2)]
_scratch += [pltpu.VMEM((_CHUNK * 2,), jnp.float32) for _ in range(2)]
_scratch += [pltpu.SemaphoreType.DMA for _ in range(4)]

_warp = functools.partial(
    pl.kernel,
    out_type=(
        jax.ShapeDtypeStruct((_B * _H * _W, _C), jnp.float32),
        jax.ShapeDtypeStruct((_B * _H * _W * 2,), jnp.float32),
    ),
    mesh=plsc.VectorSubcoreMesh(core_axis_name="c", subcore_axis_name="s",
                                num_cores=2, num_subcores=16),
    compiler_params=pltpu.CompilerParams(needs_layout_passes=False,
                                         use_tc_tiling_on_sc=False),
    scratch_types=_scratch,
)(_warp_body)


def kernel(src, disp):
    src_t = src.transpose(0, 2, 3, 1).reshape(_B * _H * _W, _C)
    dispx = disp[:, 0].reshape(_B * _H, _W)
    dispy = disp[:, 1].reshape(_B * _H, _W)
    xs = jnp.linspace(-1.0, 1.0, _W, dtype=jnp.float32)
    ys = jnp.linspace(-1.0, 1.0, _H, dtype=jnp.float32)
    out_t, flow_flat = _warp(src_t, dispx, dispy, xs, ys)
    warped = out_t.reshape(_B, _H, _W, _C).transpose(0, 3, 1, 2)
    flow = flow_flat.reshape(_B, _H, _W, 2)
    return warped, flow


# speedup vs baseline: 1.2450x; 1.0470x over previous
import functools

import jax
import jax.numpy as jnp
from jax import lax
from jax.experimental import pallas as pl
from jax.experimental.pallas import tpu as pltpu
from jax.experimental.pallas import tpu_sc as plsc

_H = 384
_W = 384
_B = 2
_C = 192
_NW = 32
_RPW = (_B * _H) // _NW
_CHUNK = 48
_GRP = _CHUNK // 16
_NCHUNK = _RPW * 8


def _floor_f32(x):
    t = x.astype(jnp.int32).astype(jnp.float32)
    return jnp.where(t > x, t - 1.0, t)


def _splat(ref, i):
    return plsc.load_gather(ref, [jnp.full((16,), i, jnp.int32)])


def _warp_body(src_t, dispx, dispy, xs, ys, out_t, flow_out, *scr):
    xs_v, ys_v, dx_v, dy_v = scr[:4]
    sets = []
    for par in range(2):
        a = 4 + par * 12
        sets.append(dict(
            idx=scr[a:a + 4], w=scr[a + 4:a + 8], r=scr[a + 8:a + 12],
            acc=scr[28 + par], flow=scr[30 + par],
            gsem=scr[32 + par], osem=scr[34 + par],
        ))

    cid = lax.axis_index("c")
    sid = lax.axis_index("s")
    wid = sid * 2 + cid
    b = wid // 16
    row0 = wid * _RPW
    y0 = row0 - b * _H
    tbase = b * (_H * _W)

    pltpu.sync_copy(xs.at[:], xs_v)
    pltpu.sync_copy(ys.at[pl.ds(y0, _RPW)], ys_v)
    pltpu.sync_copy(dispx.at[pl.ds(row0, _RPW)], dx_v)
    pltpu.sync_copy(dispy.at[pl.ds(row0, _RPW)], dy_v)

    lanes = lax.iota(jnp.int32, 16)

    def fire(q, S):
        j = q // 8
        c8 = q - j * 8
        yv = _splat(ys_v, j)
        for g in range(_GRP):
            x0 = c8 * _CHUNK + g * 16
            fx = xs_v[pl.ds(x0, 16)] + dx_v[j, pl.ds(x0, 16)]
            fy = yv + dy_v[j, pl.ds(x0, 16)]
            ix = (fx + 1.0) * 0.5 * float(_W - 1)
            iy = (fy + 1.0) * 0.5 * float(_H - 1)
            ix0 = _floor_f32(ix)
            iy0 = _floor_f32(iy)
            ix1 = ix0 + 1.0
            iy1 = iy0 + 1.0
            wx1 = ix - ix0
            wx0 = 1.0 - wx1
            wy1 = iy - iy0
            wy0 = 1.0 - wy1
            inx0 = (ix0 >= 0.0) & (ix0 <= float(_W - 1))
            inx1 = (ix1 >= 0.0) & (ix1 <= float(_W - 1))
            iny0 = (iy0 >= 0.0) & (iy0 <= float(_H - 1))
            iny1 = (iy1 >= 0.0) & (iy1 <= float(_H - 1))
            cx0 = jnp.clip(ix0, 0.0, float(_W - 1)).astype(jnp.int32)
            cx1 = jnp.clip(ix1, 0.0, float(_W - 1)).astype(jnp.int32)
            cy0 = jnp.clip(iy0, 0.0, float(_H - 1)).astype(jnp.int32) * _W + tbase
            cy1 = jnp.clip(iy1, 0.0, float(_H - 1)).astype(jnp.int32) * _W + tbase
            s = pl.ds(g * 16, 16)
            S["idx"][0][s] = cy0 + cx0
            S["idx"][1][s] = cy0 + cx1
            S["idx"][2][s] = cy1 + cx0
            S["idx"][3][s] = cy1 + cx1
            S["w"][0][s] = wy0 * wx0 * (iny0 & inx0).astype(jnp.float32)
            S["w"][1][s] = wy0 * wx1 * (iny0 & inx1).astype(jnp.float32)
            S["w"][2][s] = wy1 * wx0 * (iny1 & inx0).astype(jnp.float32)
            S["w"][3][s] = wy1 * wx1 * (iny1 & inx1).astype(jnp.float32)
            loc = (lanes + g * 16) * 2
            plsc.store_scatter(S["flow"], [loc], fx)
            plsc.store_scatter(S["flow"], [loc + 1], fy)
        for t in range(4):
            pltpu.async_copy(src_t.at[S["idx"][t]], S["r"][t], S["gsem"])

    def drain_gathers(S):
        for t in range(4):
            pltpu.make_async_copy(src_t.at[S["idx"][t]], S["r"][t],
                                  S["gsem"]).wait()

    def blend(S):
        def pix_body(i, carry):
            b00 = _splat(S["w"][0], i)
            b01 = _splat(S["w"][1], i)
            b10 = _splat(S["w"][2], i)
            b11 = _splat(S["w"][3], i)
            col = jnp.full((16,), i, jnp.int32)
            for cc in range(_C // 16):
                cs = pl.ds(cc * 16, 16)
                v = (S["r"][0][i, cs] * b00 + S["r"][1][i, cs] * b01
                     + S["r"][2][i, cs] * b10 + S["r"][3][i, cs] * b11)
                plsc.store_scatter(S["acc"], [lanes + cc * 16, col], v)
            return carry

        lax.fori_loop(0, _CHUNK, pix_body, None)

    def write_out(q, S):
        j = q // 8
        x0c = (q - j * 8) * _CHUNK
        base = (row0 + j) * _W + x0c
        pltpu.sync_copy(S["acc"], out_t.at[b, :, y0 + j, pl.ds(x0c, _CHUNK)])
        pltpu.sync_copy(S["flow"], flow_out.at[pl.ds(base * 2, _CHUNK * 2)])

    def loop_i(i, carry):
        fire(2 * i, sets[0])

        @pl.when(i > 0)
        def _():
            drain_gathers(sets[1])
            blend(sets[1])
            write_out(2 * i - 1, sets[1])

        fire(2 * i + 1, sets[1])
        drain_gathers(sets[0])
        blend(sets[0])
        write_out(2 * i, sets[0])
        return carry

    lax.fori_loop(0, _NCHUNK // 2, loop_i, None)
    drain_gathers(sets[1])
    blend(sets[1])
    write_out(_NCHUNK - 1, sets[1])


_scratch = [
    pltpu.VMEM((_W,), jnp.float32),
    pltpu.VMEM((_RPW,), jnp.float32),
    pltpu.VMEM((_RPW, _W), jnp.float32),
    pltpu.VMEM((_RPW, _W), jnp.float32),
]
for _par in range(2):
    _scratch += [pltpu.VMEM((_CHUNK,), jnp.int32) for _ in range(4)]
    _scratch += [pltpu.VMEM((_CHUNK,), jnp.float32) for _ in range(4)]
    _scratch += [pltpu.VMEM((_CHUNK, _C), jnp.float32) for _ in range(4)]
_scratch += [pltpu.VMEM((_C, _CHUNK), jnp.float32) for _ in range(2)]
_scratch += [pltpu.VMEM((_CHUNK * 2,), jnp.float32) for _ in range(2)]
_scratch += [pltpu.SemaphoreType.DMA for _ in range(4)]

_warp = functools.partial(
    pl.kernel,
    out_type=(
        jax.ShapeDtypeStruct((_B, _C, _H, _W), jnp.float32),
        jax.ShapeDtypeStruct((_B * _H * _W * 2,), jnp.float32),
    ),
    mesh=plsc.VectorSubcoreMesh(core_axis_name="c", subcore_axis_name="s",
                                num_cores=2, num_subcores=16),
    compiler_params=pltpu.CompilerParams(needs_layout_passes=False,
                                         use_tc_tiling_on_sc=False),
    scratch_types=_scratch,
)(_warp_body)


def kernel(src, disp):
    src_t = src.transpose(0, 2, 3, 1).reshape(_B * _H * _W, _C)
    dispx = disp[:, 0].reshape(_B * _H, _W)
    dispy = disp[:, 1].reshape(_B * _H, _W)
    xs = jnp.linspace(-1.0, 1.0, _W, dtype=jnp.float32)
    ys = jnp.linspace(-1.0, 1.0, _H, dtype=jnp.float32)
    warped, flow_flat = _warp(src_t, dispx, dispy, xs, ys)
    flow = flow_flat.reshape(_B, _H, _W, 2)
    return warped, flow


# speedup vs baseline: 1.3090x; 1.0514x over previous
import functools

import jax
import jax.numpy as jnp
from jax import lax
from jax.experimental import pallas as pl
from jax.experimental.pallas import tpu as pltpu
from jax.experimental.pallas import tpu_sc as plsc

_H = 384
_W = 384
_B = 2
_C = 192
_NW = 32
_RPW = (_B * _H) // _NW
_CHUNK = 48
_GRP = _CHUNK // 16
_NCHUNK = _RPW * 8


def _floor_f32(x):
    t = x.astype(jnp.int32).astype(jnp.float32)
    return jnp.where(t > x, t - 1.0, t)


def _splat(ref, i):
    return plsc.load_gather(ref, [jnp.full((16,), i, jnp.int32)])


def _warp_body(src_t, dispx, dispy, xs, ys, out_t, flow_out, *scr):
    xs_v, ys_v, dx_v, dy_v = scr[:4]
    rowflow = scr[28]
    sets = []
    for par in range(2):
        a = 4 + par * 12
        sets.append(dict(
            idx=scr[a:a + 4], w=scr[a + 4:a + 8], r=scr[a + 8:a + 12],
            acc=scr[29 + par],
            gsem=scr[31 + par], osem=scr[33 + par],
        ))

    cid = lax.axis_index("c")
    sid = lax.axis_index("s")
    wid = sid * 2 + cid
    b = wid // 16
    row0 = wid * _RPW
    y0 = row0 - b * _H
    tbase = b * (_H * _W)

    pltpu.sync_copy(xs.at[:], xs_v)
    pltpu.sync_copy(ys.at[pl.ds(y0, _RPW)], ys_v)
    pltpu.sync_copy(dispx.at[pl.ds(row0, _RPW)], dx_v)
    pltpu.sync_copy(dispy.at[pl.ds(row0, _RPW)], dy_v)

    lanes = lax.iota(jnp.int32, 16)

    def fire(q, S):
        j = q // 8
        c8 = q - j * 8
        yv = _splat(ys_v, j)
        for g in range(_GRP):
            x0 = c8 * _CHUNK + g * 16
            fx = xs_v[pl.ds(x0, 16)] + dx_v[j, pl.ds(x0, 16)]
            fy = yv + dy_v[j, pl.ds(x0, 16)]
            ix = (fx + 1.0) * 0.5 * float(_W - 1)
            iy = (fy + 1.0) * 0.5 * float(_H - 1)
            ix0 = _floor_f32(ix)
            iy0 = _floor_f32(iy)
            ix1 = ix0 + 1.0
            iy1 = iy0 + 1.0
            wx1 = ix - ix0
            wx0 = 1.0 - wx1
            wy1 = iy - iy0
            wy0 = 1.0 - wy1
            inx0 = (ix0 >= 0.0) & (ix0 <= float(_W - 1))
            inx1 = (ix1 >= 0.0) & (ix1 <= float(_W - 1))
            iny0 = (iy0 >= 0.0) & (iy0 <= float(_H - 1))
            iny1 = (iy1 >= 0.0) & (iy1 <= float(_H - 1))
            cx0 = jnp.clip(ix0, 0.0, float(_W - 1)).astype(jnp.int32)
            cx1 = jnp.clip(ix1, 0.0, float(_W - 1)).astype(jnp.int32)
            cy0 = jnp.clip(iy0, 0.0, float(_H - 1)).astype(jnp.int32) * _W + tbase
            cy1 = jnp.clip(iy1, 0.0, float(_H - 1)).astype(jnp.int32) * _W + tbase
            s = pl.ds(g * 16, 16)
            S["idx"][0][s] = cy0 + cx0
            S["idx"][1][s] = cy0 + cx1
            S["idx"][2][s] = cy1 + cx0
            S["idx"][3][s] = cy1 + cx1
            S["w"][0][s] = wy0 * wx0 * (iny0 & inx0).astype(jnp.float32)
            S["w"][1][s] = wy0 * wx1 * (iny0 & inx1).astype(jnp.float32)
            S["w"][2][s] = wy1 * wx0 * (iny1 & inx0).astype(jnp.float32)
            S["w"][3][s] = wy1 * wx1 * (iny1 & inx1).astype(jnp.float32)
            loc = (lanes + x0) * 2
            plsc.store_scatter(rowflow, [loc], fx)
            plsc.store_scatter(rowflow, [loc + 1], fy)
        for t in range(4):
            pltpu.async_copy(src_t.at[S["idx"][t]], S["r"][t], S["gsem"])

    def drain_gathers(S):
        for t in range(4):
            pltpu.make_async_copy(src_t.at[S["idx"][t]], S["r"][t],
                                  S["gsem"]).wait()

    def blend(S):
        def pix_body(i, carry):
            b00 = _splat(S["w"][0], i)
            b01 = _splat(S["w"][1], i)
            b10 = _splat(S["w"][2], i)
            b11 = _splat(S["w"][3], i)
            col = jnp.full((16,), i, jnp.int32)
            for cc in range(_C // 16):
                cs = pl.ds(cc * 16, 16)
                v = (S["r"][0][i, cs] * b00 + S["r"][1][i, cs] * b01
                     + S["r"][2][i, cs] * b10 + S["r"][3][i, cs] * b11)
                plsc.store_scatter(S["acc"], [lanes + cc * 16, col], v)
            return carry

        lax.fori_loop(0, _CHUNK, pix_body, None)

    def out_dst(q):
        j = q // 8
        x0c = (q - j * 8) * _CHUNK
        return out_t.at[b, :, y0 + j, pl.ds(x0c, _CHUNK)]

    def start_out(q, S):
        pltpu.async_copy(S["acc"], out_dst(q), S["osem"])

    def drain_out(q, S):
        pltpu.make_async_copy(S["acc"], out_dst(q), S["osem"]).wait()

    def flush_rowflow(q):
        j = q // 8
        base = (row0 + j) * _W
        pltpu.sync_copy(rowflow, flow_out.at[pl.ds(base * 2, _W * 2)])

    def loop_i(i, carry):
        fire(2 * i, sets[0])

        @pl.when(i > 1)
        def _():
            drain_out(2 * i - 3, sets[1])

        @pl.when(i > 0)
        def _():
            drain_gathers(sets[1])
            blend(sets[1])
            start_out(2 * i - 1, sets[1])

        fire(2 * i + 1, sets[1])
        drain_gathers(sets[0])

        @pl.when(i > 0)
        def _():
            drain_out(2 * i - 2, sets[0])

        blend(sets[0])
        start_out(2 * i, sets[0])

        @pl.when((2 * i + 1) % 8 == 7)
        def _():
            flush_rowflow(2 * i + 1)
        return carry

    lax.fori_loop(0, _NCHUNK // 2, loop_i, None)
    drain_gathers(sets[1])
    drain_out(_NCHUNK - 3, sets[1])
    blend(sets[1])
    pltpu.sync_copy(sets[1]["acc"], out_dst(_NCHUNK - 1))
    drain_out(_NCHUNK - 2, sets[0])


_scratch = [
    pltpu.VMEM((_W,), jnp.float32),
    pltpu.VMEM((_RPW,), jnp.float32),
    pltpu.VMEM((_RPW, _W), jnp.float32),
    pltpu.VMEM((_RPW, _W), jnp.float32),
]
for _par in range(2):
    _scratch += [pltpu.VMEM((_CHUNK,), jnp.int32) for _ in range(4)]
    _scratch += [pltpu.VMEM((_CHUNK,), jnp.float32) for _ in range(4)]
    _scratch += [pltpu.VMEM((_CHUNK, _C), jnp.float32) for _ in range(4)]
_scratch += [pltpu.VMEM((_W * 2,), jnp.float32)]
_scratch += [pltpu.VMEM((_C, _CHUNK), jnp.float32) for _ in range(2)]
_scratch += [pltpu.SemaphoreType.DMA for _ in range(4)]

_warp = functools.partial(
    pl.kernel,
    out_type=(
        jax.ShapeDtypeStruct((_B, _C, _H, _W), jnp.float32),
        jax.ShapeDtypeStruct((_B * _H * _W * 2,), jnp.float32),
    ),
    mesh=plsc.VectorSubcoreMesh(core_axis_name="c", subcore_axis_name="s",
                                num_cores=2, num_subcores=16),
    compiler_params=pltpu.CompilerParams(needs_layout_passes=False,
                                         use_tc_tiling_on_sc=False),
    scratch_types=_scratch,
)(_warp_body)


def kernel(src, disp):
    src_t = src.transpose(0, 2, 3, 1).reshape(_B * _H * _W, _C)
    dispx = disp[:, 0].reshape(_B * _H, _W)
    dispy = disp[:, 1].reshape(_B * _H, _W)
    xs = jnp.linspace(-1.0, 1.0, _W, dtype=jnp.float32)
    ys = jnp.linspace(-1.0, 1.0, _H, dtype=jnp.float32)
    warped, flow_flat = _warp(src_t, dispx, dispy, xs, ys)
    flow = flow_flat.reshape(_B, _H, _W, 2)
    return warped, flow
